# baseline (speedup 1.0000x reference)
import functools

import jax
import jax.numpy as jnp
from jax import lax
from jax.experimental import pallas as pl
from jax.experimental.pallas import tpu as pltpu
from jax.experimental.pallas import tpu_sc as plsc

D = 64
SCALE = 0.125
NUM_WORKERS = 32
CHUNK = 1024


def kernel(inputs, table):
    idx = inputs.reshape(-1)
    n = idx.shape[0]
    per_worker = n // NUM_WORKERS
    steps = per_worker // CHUNK
    assert per_worker % CHUNK == 0 and n % NUM_WORKERS == 0

    mesh = plsc.VectorSubcoreMesh(core_axis_name="c", subcore_axis_name="s")

    @functools.partial(
        pl.kernel,
        out_type=jax.ShapeDtypeStruct((n, D), jnp.float32),
        mesh=mesh,
        compiler_params=pltpu.CompilerParams(use_tc_tiling_on_sc=False),
        scratch_types=[
            pltpu.VMEM((CHUNK,), jnp.int32),
            pltpu.VMEM((CHUNK, D), jnp.float32),
            pltpu.SemaphoreType.DMA,
        ],
    )
    def emb(table_hbm, idx_hbm, out_hbm, idx_v, rows_v, sem):
        wid = lax.axis_index("s") * 2 + lax.axis_index("c")
        base = wid * per_worker

        @pl.loop(0, steps)
        def _(s):
            off = base + s * CHUNK
            pltpu.sync_copy(idx_hbm.at[pl.ds(off, CHUNK)], idx_v)
            pltpu.async_copy(table_hbm.at[idx_v], rows_v, sem).wait()

            @pl.loop(0, CHUNK)
            def _(j):
                row = rows_v.at[j]
                for c in range(0, D, 16):
                    row[pl.ds(c, 16)] = row[pl.ds(c, 16)] * SCALE

            pltpu.sync_copy(rows_v, out_hbm.at[pl.ds(off, CHUNK)])

    out = emb(table, idx)
    return out.reshape(inputs.shape + (D,))


# speedup vs baseline: 1.1106x; 1.1106x over previous
import functools

import jax
import jax.numpy as jnp
from jax import lax
from jax.experimental import pallas as pl
from jax.experimental.pallas import tpu as pltpu
from jax.experimental.pallas import tpu_sc as plsc

D = 64
SCALE = 0.125
NUM_WORKERS = 32
CHUNK = 400
ROW_UNROLL = 4


def kernel(inputs, table):
    idx = inputs.reshape(-1)
    n = idx.shape[0]
    per_worker = n // NUM_WORKERS
    steps = per_worker // CHUNK
    assert n % NUM_WORKERS == 0 and per_worker % CHUNK == 0 and steps % 2 == 0

    mesh = plsc.VectorSubcoreMesh(core_axis_name="c", subcore_axis_name="s")

    @functools.partial(
        pl.kernel,
        out_type=jax.ShapeDtypeStruct((n, D), jnp.float32),
        mesh=mesh,
        compiler_params=pltpu.CompilerParams(use_tc_tiling_on_sc=False),
        scratch_types=[
            pltpu.VMEM((per_worker,), jnp.int32),
            pltpu.VMEM((CHUNK, D), jnp.float32),
            pltpu.VMEM((CHUNK, D), jnp.float32),
            pltpu.VMEM((CHUNK, D), jnp.float32),
            pltpu.VMEM((CHUNK, D), jnp.float32),
            pltpu.SemaphoreType.DMA,
            pltpu.SemaphoreType.DMA,
            pltpu.SemaphoreType.DMA,
            pltpu.SemaphoreType.DMA,
        ],
    )
    def emb(table_hbm, idx_hbm, out_hbm, idx_v, g0, g1, o0, o1,
            gsem0, gsem1, osem0, osem1):
        gbuf = (g0, g1)
        obuf = (o0, o1)
        gsem = (gsem0, gsem1)
        osem = (osem0, osem1)
        wid = lax.axis_index("s") * 2 + lax.axis_index("c")
        base = wid * per_worker

        pltpu.sync_copy(idx_hbm.at[pl.ds(base, per_worker)], idx_v)

        def gather_start(cur, b):
            src = table_hbm.at[idx_v.at[pl.ds(cur * CHUNK, CHUNK)]]
            pltpu.make_async_copy(src, gbuf[b], gsem[b]).start()

        def gather_wait(cur, b):
            src = table_hbm.at[idx_v.at[pl.ds(cur * CHUNK, CHUNK)]]
            pltpu.make_async_copy(src, gbuf[b], gsem[b]).wait()

        def out_start(cur, b):
            dst = out_hbm.at[pl.ds(base + cur * CHUNK, CHUNK)]
            pltpu.make_async_copy(obuf[b], dst, osem[b]).start()

        def out_wait(cur, b):
            dst = out_hbm.at[pl.ds(base + cur * CHUNK, CHUNK)]
            pltpu.make_async_copy(obuf[b], dst, osem[b]).wait()

        gather_start(0, 0)
        gather_start(1, 1)

        @pl.loop(0, steps, step=2)
        def _(s):
            for b in range(2):
                cur = s + b
                gather_wait(cur, b)

                @pl.when(cur >= 2)
                def _():
                    out_wait(cur - 2, b)

                @pl.loop(0, CHUNK, step=ROW_UNROLL)
                def _(j):
                    for r in range(ROW_UNROLL):
                        src = gbuf[b].at[j + r]
                        dst = obuf[b].at[j + r]
                        for c in range(0, D, 16):
                            dst[pl.ds(c, 16)] = src[pl.ds(c, 16)] * SCALE

                out_start(cur, b)

                @pl.when(cur + 2 < steps)
                def _():
                    gather_start(cur + 2, b)

        out_wait(steps - 2, 0)
        out_wait(steps - 1, 1)

    out = emb(table, idx)
    return out.reshape(inputs.shape + (D,))
